# baseline (speedup 1.0000x reference)
import jax
import jax.numpy as jnp
from jax import lax
from jax.experimental import pallas as pl
from jax.experimental.pallas import tpu as pltpu
from jax.experimental.pallas import tpu_sc as plsc

VOCAB = 1000000
EMBED_DIM = 32
BATCH = 16384
HIST = 200

NC = 2
NS = 16
NW = NC * NS

N = BATCH * HIST
PER_W = N // NW
CH = 1600
STEPS = PER_W // CH


def _fire(x_hbm, table_hbm, idx_v, rows_v, sems, wid, s, slot):
    pltpu.sync_copy(x_hbm.at[wid, s], idx_v.at[slot])
    pltpu.async_copy(
        table_hbm.at[idx_v.at[slot]],
        rows_v.at[slot],
        sems.at[slot],
    )


def _drain(table_hbm, idx_v, rows_v, sems, slot):
    pltpu.make_async_copy(
        table_hbm.at[idx_v.at[slot]],
        rows_v.at[slot],
        sems.at[slot],
    ).wait()


def _body(x_hbm, table_hbm, out_hbm, idx_v, rows_v, sems):
    wid = lax.axis_index("s") * NC + lax.axis_index("c")
    base = wid * PER_W

    _fire(x_hbm, table_hbm, idx_v, rows_v, sems, wid, 0, 0)

    def pair_fn(g, carry):
        for b in range(2):
            s = 2 * g + b

            @pl.when(s + 1 < STEPS)
            def _():
                _fire(x_hbm, table_hbm, idx_v, rows_v, sems, wid, s + 1, 1 - b)

            _drain(table_hbm, idx_v, rows_v, sems, b)
            pltpu.sync_copy(rows_v.at[b], out_hbm.at[pl.ds(base + s * CH, CH)])
        return carry

    lax.fori_loop(0, STEPS // 2, pair_fn, 0)


@jax.jit
def _lookup(x_r, table_bf):
    mesh = plsc.VectorSubcoreMesh(core_axis_name="c", subcore_axis_name="s")
    f = pl.kernel(
        _body,
        out_type=jax.ShapeDtypeStruct((N, EMBED_DIM), jnp.bfloat16),
        mesh=mesh,
        scratch_types=[
            pltpu.VMEM((2, CH), jnp.int32),
            pltpu.VMEM((2, CH, EMBED_DIM), jnp.bfloat16),
            pltpu.SemaphoreType.DMA((2,)),
        ],
        compiler_params=pltpu.CompilerParams(use_tc_tiling_on_sc=False),
    )
    return f(x_r, table_bf)


def kernel(x, table):
    x_r = x.reshape(NW, STEPS, CH).astype(jnp.int32)
    table_bf = table.astype(jnp.bfloat16)
    out = _lookup(x_r, table_bf)
    return out.astype(jnp.float32).reshape(BATCH, HIST, EMBED_DIM)


# speedup vs baseline: 1.4711x; 1.4711x over previous
import jax
import jax.numpy as jnp
from jax import lax
from jax.experimental import pallas as pl
from jax.experimental.pallas import tpu as pltpu
from jax.experimental.pallas import tpu_sc as plsc

VOCAB = 1000000
EMBED_DIM = 32
BATCH = 16384
HIST = 200

NC = 2
NS = 16
NW = NC * NS

N = BATCH * HIST
PER_W = N // NW
CH = 800
NBUF = 4
STEPS = PER_W // CH


def _body(x_hbm, table_hbm, out_hbm, idx_v, rows_v, isem, gsem, ssem):
    wid = lax.axis_index("s") * NC + lax.axis_index("c")
    base = wid * PER_W

    def idx_start(s, slot):
        pltpu.async_copy(x_hbm.at[wid, s], idx_v.at[slot], isem.at[slot])

    def idx_wait(s, slot):
        pltpu.make_async_copy(
            x_hbm.at[wid, s], idx_v.at[slot], isem.at[slot]
        ).wait()

    def gather_start(slot):
        pltpu.async_copy(
            table_hbm.at[idx_v.at[slot]], rows_v.at[slot], gsem.at[slot]
        )

    def gather_wait(slot):
        pltpu.make_async_copy(
            table_hbm.at[idx_v.at[slot]], rows_v.at[slot], gsem.at[slot]
        ).wait()

    def store_start(s, slot):
        pltpu.async_copy(
            rows_v.at[slot], out_hbm.at[pl.ds(base + s * CH, CH)], ssem.at[slot]
        )

    def store_wait(s, slot):
        pltpu.make_async_copy(
            rows_v.at[slot], out_hbm.at[pl.ds(base + s * CH, CH)], ssem.at[slot]
        ).wait()

    pltpu.sync_copy(x_hbm.at[wid, 0], idx_v.at[0])
    gather_start(0)
    pltpu.sync_copy(x_hbm.at[wid, 1], idx_v.at[1])
    gather_start(1)

    def group_fn(g, carry):
        for b in range(NBUF):
            s = 4 * g + b
            nslot = (b + 2) % NBUF

            gather_wait(b)
            store_start(s, b)

            @pl.when(s + 2 < STEPS)
            def _():
                @pl.when(s >= 2)
                def _():
                    store_wait(s - 2, nslot)

                pltpu.sync_copy(x_hbm.at[wid, s + 2], idx_v.at[nslot])
                gather_start(nslot)

        return carry

    lax.fori_loop(0, STEPS // NBUF, group_fn, 0)
    for k in range(NBUF, 0, -1):
        store_wait(STEPS - k, (STEPS - k) % NBUF)


@jax.jit
def _lookup(x_r, table):
    mesh = plsc.VectorSubcoreMesh(core_axis_name="c", subcore_axis_name="s")
    f = pl.kernel(
        _body,
        out_type=jax.ShapeDtypeStruct((N, EMBED_DIM), jnp.float32),
        mesh=mesh,
        scratch_types=[
            pltpu.VMEM((NBUF, CH), jnp.int32),
            pltpu.VMEM((NBUF, CH, EMBED_DIM), jnp.float32),
            pltpu.SemaphoreType.DMA((NBUF,)),
            pltpu.SemaphoreType.DMA((NBUF,)),
            pltpu.SemaphoreType.DMA((NBUF,)),
        ],
        compiler_params=pltpu.CompilerParams(use_tc_tiling_on_sc=False),
    )
    return f(x_r, table)


def kernel(x, table):
    x_r = x.reshape(NW, STEPS, CH).astype(jnp.int32)
    out = _lookup(x_r, table)
    return out.reshape(BATCH, HIST, EMBED_DIM)


# speedup vs baseline: 1.4712x; 1.0001x over previous
import jax
import jax.numpy as jnp
from jax import lax
from jax.experimental import pallas as pl
from jax.experimental.pallas import tpu as pltpu
from jax.experimental.pallas import tpu_sc as plsc

VOCAB = 1000000
EMBED_DIM = 32
BATCH = 16384
HIST = 200

NC = 2
NS = 16
NW = NC * NS

N = BATCH * HIST
PER_W = N // NW
CH = 800
NBUF = 4
STEPS = PER_W // CH


def _body(x_hbm, table_hbm, out_hbm, idx_v, rows_v, gsem, ssem):
    wid = lax.axis_index("s") * NC + lax.axis_index("c")
    base = wid * PER_W

    def gather_start(slot):
        pltpu.async_copy(
            table_hbm.at[idx_v.at[slot]], rows_v.at[slot], gsem.at[slot]
        )

    def gather_wait(slot):
        pltpu.make_async_copy(
            table_hbm.at[idx_v.at[slot]], rows_v.at[slot], gsem.at[slot]
        ).wait()

    def store_start(s, slot):
        pltpu.async_copy(
            rows_v.at[slot], out_hbm.at[pl.ds(base + s * CH, CH)], ssem.at[slot]
        )

    def store_wait(s, slot):
        pltpu.make_async_copy(
            rows_v.at[slot], out_hbm.at[pl.ds(base + s * CH, CH)], ssem.at[slot]
        ).wait()

    pltpu.sync_copy(x_hbm.at[wid, 0], idx_v.at[0])
    gather_start(0)
    pltpu.sync_copy(x_hbm.at[wid, 1], idx_v.at[1])
    gather_start(1)

    def group_fn(g, carry):
        for b in range(NBUF):
            s = 4 * g + b
            nslot = (b + 2) % NBUF

            gather_wait(b)
            store_start(s, b)

            @pl.when(s + 2 < STEPS)
            def _():
                @pl.when(s >= 2)
                def _():
                    store_wait(s - 2, nslot)

                pltpu.sync_copy(x_hbm.at[wid, s + 2], idx_v.at[nslot])
                gather_start(nslot)

        return carry

    lax.fori_loop(0, STEPS // NBUF, group_fn, 0)
    for k in range(NBUF, 0, -1):
        store_wait(STEPS - k, (STEPS - k) % NBUF)


@jax.jit
def _lookup(x_r, table):
    mesh = plsc.VectorSubcoreMesh(core_axis_name="c", subcore_axis_name="s")
    f = pl.kernel(
        _body,
        out_type=jax.ShapeDtypeStruct((N, EMBED_DIM), jnp.float32),
        mesh=mesh,
        scratch_types=[
            pltpu.VMEM((NBUF, CH), jnp.int32),
            pltpu.VMEM((NBUF, CH, EMBED_DIM), jnp.float32),
            pltpu.SemaphoreType.DMA((NBUF,)),
            pltpu.SemaphoreType.DMA((NBUF,)),
        ],
        compiler_params=pltpu.CompilerParams(use_tc_tiling_on_sc=False),
    )
    return f(x_r, table)


def kernel(x, table):
    x_r = x.reshape(NW, STEPS, CH).astype(jnp.int32)
    out = _lookup(x_r, table)
    return out.reshape(BATCH, HIST, EMBED_DIM)
